# baseline (speedup 1.0000x reference)
import functools

import jax
import jax.numpy as jnp
from jax import lax
from jax.experimental import pallas as pl
from jax.experimental.pallas import tpu as pltpu
from jax.experimental.pallas import tpu_sc as plsc

A, B, I, C, J, K = 8, 16, 64, 32, 16, 8
T = 64
NC, NS = 2, 16
NW = NC * NS
NBUF = 4
BQ = B // NBUF
TTC = 32
TSC = T - TTC


def _scalar_at(ref, pos):
    v = plsc.load_gather(ref, [jnp.broadcast_to(pos, (16,))])
    return jnp.max(v)


def _sc_body(xp, idx_all, out, idxv, gb0, gb1, gb2, gb3,
             ob0, ob1, ob2, ob3, s0, s1, s2, s3):
    wid = lax.axis_index("s") * NC + lax.axis_index("c")
    pltpu.sync_copy(idx_all, idxv)

    iota = lax.iota(jnp.int32, 16)
    wa = jnp.bitwise_and(lax.shift_right_logical(wid, 2), 7)
    wq = jnp.bitwise_and(wid, 3)
    wb = wq * BQ

    def t_indices(t):
        i1t = _scalar_at(idxv, t)
        i2t = _scalar_at(idxv, T + lax.shift_right_logical(t, 3))
        i3t = _scalar_at(idxv, T + A + jnp.bitwise_and(t, 7))
        return i1t, i2t, i3t

    tparams = [t_indices(TTC + m) for m in range(TSC)]

    def extract(gb, ob, i1t):
        lanev = jnp.broadcast_to(i1t, (16,))
        for q in range(8):
            v = plsc.load_gather(
                gb, [jnp.broadcast_to(q >> 1, (16,)),
                     (q & 1) * 16 + iota, lanev])
            ob[q >> 1, pl.ds((q & 1) * 16, 16)] = v

    gbufs = (gb0, gb1, gb2, gb3)
    obufs = (ob0, ob1, ob2, ob3)
    sems = (s0, s1, s2, s3)

    def fire(m):
        s = m % NBUF
        _, i2t, i3t = tparams[m]
        return pltpu.async_copy(
            xp.at[wa, pl.ds(wb, BQ), :, i2t, i3t, :], gbufs[s], sems[s])

    def retire(m, cp):
        s = m % NBUF
        cp.wait()
        extract(gbufs[s], obufs[s], tparams[m][0])
        pltpu.sync_copy(obufs[s], out.at[m, wa, pl.ds(wb, BQ)])

    pend = [None] * NBUF
    for m in range(TSC):
        if m >= NBUF - 1:
            p = m - (NBUF - 1)
            if pend[p % NBUF] is not None:
                retire(p, pend[p % NBUF])
                pend[p % NBUF] = None
        pend[m % NBUF] = fire(m)
    for p in range(max(0, TSC - (NBUF - 1)), TSC):
        if pend[p % NBUF] is not None:
            retire(p, pend[p % NBUF])
            pend[p % NBUF] = None


_sc_gather = functools.partial(
    pl.kernel,
    out_type=jax.ShapeDtypeStruct((TSC, A, B, C), jnp.float32),
    mesh=plsc.VectorSubcoreMesh(
        core_axis_name="c", subcore_axis_name="s",
        num_cores=NC, num_subcores=NS),
    compiler_params=pltpu.CompilerParams(needs_layout_passes=False),
    cost_estimate=pl.CostEstimate(
        flops=0, bytes_accessed=70_000_000, transcendentals=0),
    scratch_types=[
        pltpu.VMEM((T + A + K,), jnp.int32),
        pltpu.VMEM((BQ, C, I), jnp.float32),
        pltpu.VMEM((BQ, C, I), jnp.float32),
        pltpu.VMEM((BQ, C, I), jnp.float32),
        pltpu.VMEM((BQ, C, I), jnp.float32),
        pltpu.VMEM((BQ, C), jnp.float32),
        pltpu.VMEM((BQ, C), jnp.float32),
        pltpu.VMEM((BQ, C), jnp.float32),
        pltpu.VMEM((BQ, C), jnp.float32),
        pltpu.SemaphoreType.DMA,
        pltpu.SemaphoreType.DMA,
        pltpu.SemaphoreType.DMA,
        pltpu.SemaphoreType.DMA,
    ],
)(_sc_body)


def _tc_body(idx_ref, x_ref, oh1_ref, m2_ref, o_ref):
    g = pl.program_id(1)
    xm = x_ref[...].reshape(B * C * K, I)
    y = jnp.dot(xm, oh1_ref[...], precision=lax.Precision.HIGHEST,
                preferred_element_type=jnp.float32)
    yv = y.reshape(B * C, K, TTC)
    tg = lax.broadcasted_iota(jnp.int32, (K, TTC), 1) >> 3
    m = jnp.where(tg == g, m2_ref[...], 0.0)
    z = jnp.sum(yv * m[None], axis=1)

    @pl.when(g == 0)
    def _():
        o_ref[...] = jnp.zeros_like(o_ref)

    o_ref[0] += z


def _tc_gather(xv, idx_all, oh1, m2):
    g_spec = pltpu.PrefetchScalarGridSpec(
        num_scalar_prefetch=1,
        grid=(A, TTC // K),
        in_specs=[
            pl.BlockSpec((1, B, C, K, I),
                         lambda a, g, idx: (a, 0, 0, idx[T + g], 0)),
            pl.BlockSpec((I, TTC), lambda a, g, idx: (0, 0)),
            pl.BlockSpec((K, TTC), lambda a, g, idx: (0, 0)),
        ],
        out_specs=pl.BlockSpec((1, B * C, TTC), lambda a, g, idx: (a, 0, 0)),
    )
    return pl.pallas_call(
        _tc_body,
        grid_spec=g_spec,
        out_shape=jax.ShapeDtypeStruct((A, B * C, TTC), jnp.float32),
        compiler_params=pltpu.CompilerParams(
            dimension_semantics=("arbitrary", "arbitrary")),
    )(idx_all, xv, oh1, m2)


def kernel(x, index1, index2, index3):
    xp = jnp.transpose(x, (0, 1, 3, 4, 5, 2))
    idx_all = jnp.concatenate([
        index1.reshape(T).astype(jnp.int32),
        index2.reshape(A).astype(jnp.int32),
        index3.reshape(K).astype(jnp.int32),
    ])
    parts = []
    if TTC:
        xv = xp.reshape(A, B, C, J * K, I)
        i1 = idx_all[:TTC]
        i3b = jnp.tile(idx_all[T + A:], TTC // K)[:TTC]
        oh1 = (jnp.arange(I, dtype=jnp.int32)[:, None] == i1[None, :]
               ).astype(jnp.float32)
        m2 = (jnp.arange(K, dtype=jnp.int32)[:, None] == i3b[None, :]
              ).astype(jnp.float32)
        tc = _tc_gather(xv, idx_all, oh1, m2)
        parts.append(jnp.transpose(tc, (2, 0, 1)).reshape(TTC, A, B, C))
    if TSC:
        parts.append(_sc_gather(xp, idx_all))
    out = parts[0] if len(parts) == 1 else jnp.concatenate(parts, axis=0)
    return out.reshape(A, 4, 2, A, B, C)


# speedup vs baseline: 1.2700x; 1.2700x over previous
import functools

import jax
import jax.numpy as jnp
from jax import lax
from jax.experimental import pallas as pl
from jax.experimental.pallas import tpu as pltpu
from jax.experimental.pallas import tpu_sc as plsc

A, B, I, C, J, K = 8, 16, 64, 32, 16, 8
T = 64
NC, NS = 2, 16
NW = NC * NS
NBUF = 4
BQ = B // NBUF
TTC = 32
TSC = T - TTC


def _scalar_at(ref, pos):
    v = plsc.load_gather(ref, [jnp.broadcast_to(pos, (16,))])
    return jnp.max(v)


def _sc_body(xp, idx_all, out, idxv, gb0, gb1, gb2, gb3,
             ob0, ob1, ob2, ob3, s0, s1, s2, s3):
    wid = lax.axis_index("s") * NC + lax.axis_index("c")
    pltpu.sync_copy(idx_all, idxv)

    iota = lax.iota(jnp.int32, 16)
    wa = jnp.bitwise_and(lax.shift_right_logical(wid, 2), 7)
    wq = jnp.bitwise_and(wid, 3)
    wb = wq * BQ

    def t_indices(t):
        i1t = _scalar_at(idxv, t)
        i2t = _scalar_at(idxv, T + lax.shift_right_logical(t, 3))
        i3t = _scalar_at(idxv, T + A + jnp.bitwise_and(t, 7))
        return i1t, i2t, i3t

    tparams = [t_indices(TTC + m) for m in range(TSC)]

    def extract(gb, ob, i1t):
        lanev = jnp.broadcast_to(i1t, (16,))
        for q in range(8):
            v = plsc.load_gather(
                gb, [jnp.broadcast_to(q >> 1, (16,)),
                     (q & 1) * 16 + iota, lanev])
            ob[q >> 1, pl.ds((q & 1) * 16, 16)] = v

    gbufs = (gb0, gb1, gb2, gb3)
    obufs = (ob0, ob1, ob2, ob3)
    sems = (s0, s1, s2, s3)

    def fire(m):
        s = m % NBUF
        _, i2t, i3t = tparams[m]
        return pltpu.async_copy(
            xp.at[wa, pl.ds(wb, BQ), :, i2t, i3t, :], gbufs[s], sems[s])

    def retire(m, cp):
        s = m % NBUF
        cp.wait()
        extract(gbufs[s], obufs[s], tparams[m][0])
        pltpu.sync_copy(obufs[s], out.at[m, wa, pl.ds(wb, BQ)])

    pend = [None] * NBUF
    for m in range(TSC):
        if m >= NBUF - 1:
            p = m - (NBUF - 1)
            if pend[p % NBUF] is not None:
                retire(p, pend[p % NBUF])
                pend[p % NBUF] = None
        pend[m % NBUF] = fire(m)
    for p in range(max(0, TSC - (NBUF - 1)), TSC):
        if pend[p % NBUF] is not None:
            retire(p, pend[p % NBUF])
            pend[p % NBUF] = None


_sc_gather = functools.partial(
    pl.kernel,
    out_type=jax.ShapeDtypeStruct((TSC, A, B, C), jnp.float32),
    mesh=plsc.VectorSubcoreMesh(
        core_axis_name="c", subcore_axis_name="s",
        num_cores=NC, num_subcores=NS),
    compiler_params=pltpu.CompilerParams(needs_layout_passes=False),
    cost_estimate=pl.CostEstimate(
        flops=0, bytes_accessed=70_000_000, transcendentals=0),
    scratch_types=[
        pltpu.VMEM((T + A + K,), jnp.int32),
        pltpu.VMEM((BQ, C, I), jnp.float32),
        pltpu.VMEM((BQ, C, I), jnp.float32),
        pltpu.VMEM((BQ, C, I), jnp.float32),
        pltpu.VMEM((BQ, C, I), jnp.float32),
        pltpu.VMEM((BQ, C), jnp.float32),
        pltpu.VMEM((BQ, C), jnp.float32),
        pltpu.VMEM((BQ, C), jnp.float32),
        pltpu.VMEM((BQ, C), jnp.float32),
        pltpu.SemaphoreType.DMA,
        pltpu.SemaphoreType.DMA,
        pltpu.SemaphoreType.DMA,
        pltpu.SemaphoreType.DMA,
    ],
)(_sc_body)


def _tc_body(idx_ref, x_ref, oh1_ref, m2_ref, o_ref):
    g = pl.program_id(1)
    xm = x_ref[...].reshape(B * C * K, I)
    y = jnp.dot(xm, oh1_ref[...],
                preferred_element_type=jnp.float32)
    yv = y.reshape(B * C, K, TTC)
    tg = lax.broadcasted_iota(jnp.int32, (K, TTC), 1) >> 3
    m = jnp.where(tg == g, m2_ref[...], 0.0)
    z = jnp.sum(yv * m[None], axis=1)

    @pl.when(g == 0)
    def _():
        o_ref[...] = jnp.zeros_like(o_ref)

    o_ref[0] += z


def _tc_gather(xv, idx_all, oh1, m2):
    g_spec = pltpu.PrefetchScalarGridSpec(
        num_scalar_prefetch=1,
        grid=(A, TTC // K),
        in_specs=[
            pl.BlockSpec((1, B, C, K, I),
                         lambda a, g, idx: (a, 0, 0, idx[T + g], 0)),
            pl.BlockSpec((I, TTC), lambda a, g, idx: (0, 0)),
            pl.BlockSpec((K, TTC), lambda a, g, idx: (0, 0)),
        ],
        out_specs=pl.BlockSpec((1, B * C, TTC), lambda a, g, idx: (a, 0, 0)),
    )
    return pl.pallas_call(
        _tc_body,
        grid_spec=g_spec,
        out_shape=jax.ShapeDtypeStruct((A, B * C, TTC), jnp.float32),
        compiler_params=pltpu.CompilerParams(
            dimension_semantics=("arbitrary", "arbitrary")),
    )(idx_all, xv, oh1, m2)


def kernel(x, index1, index2, index3):
    xp = jnp.transpose(x, (0, 1, 3, 4, 5, 2))
    idx_all = jnp.concatenate([
        index1.reshape(T).astype(jnp.int32),
        index2.reshape(A).astype(jnp.int32),
        index3.reshape(K).astype(jnp.int32),
    ])
    parts = []
    if TTC:
        xv = xp.reshape(A, B, C, J * K, I)
        i1 = idx_all[:TTC]
        i3b = jnp.tile(idx_all[T + A:], TTC // K)[:TTC]
        oh1 = (jnp.arange(I, dtype=jnp.int32)[:, None] == i1[None, :]
               ).astype(jnp.float32)
        m2 = (jnp.arange(K, dtype=jnp.int32)[:, None] == i3b[None, :]
              ).astype(jnp.float32)
        tc = _tc_gather(xv, idx_all, oh1, m2)
        parts.append(jnp.transpose(tc, (2, 0, 1)).reshape(TTC, A, B, C))
    if TSC:
        parts.append(_sc_gather(xp, idx_all))
    out = parts[0] if len(parts) == 1 else jnp.concatenate(parts, axis=0)
    return out.reshape(A, 4, 2, A, B, C)
